# baseline (speedup 1.0000x reference)
import jax
import jax.numpy as jnp
from jax.experimental import pallas as pl
from jax.experimental.pallas import tpu as pltpu

BT = 256
BB = 4


def _scan_body(a_ref, b_ref, thr_ref, h0_ref, h_out_ref, s_out_ref, h_scr):
    t_blk = pl.program_id(1)

    @pl.when(t_blk == 0)
    def _():
        h_scr[...] = h0_ref[0]

    thr = thr_ref[...]
    h = h_scr[...]
    for i in range(BT):
        h_pre = a_ref[:, i, :] * h + b_ref[:, i, :]
        spike = h_pre > thr
        h = jnp.where(spike, 0.0, h_pre)
        h_out_ref[:, i, :] = h
        s_out_ref[:, i, :] = jnp.where(spike, 1.0, 0.0)
    h_scr[...] = h


def kernel(a, b, threshold, h0):
    B, T, D = a.shape
    thr2d = threshold.reshape(1, D)
    h0_blocked = h0.reshape(B // BB, BB, D)

    grid = (B // BB, T // BT)
    blk_btd = pl.BlockSpec((BB, BT, D), lambda p, t: (p, t, 0))

    h_post, s = pl.pallas_call(
        _scan_body,
        out_shape=(
            jax.ShapeDtypeStruct((B, T, D), jnp.float32),
            jax.ShapeDtypeStruct((B, T, D), jnp.float32),
        ),
        grid=grid,
        in_specs=[
            blk_btd,
            blk_btd,
            pl.BlockSpec((1, D), lambda p, t: (0, 0)),
            pl.BlockSpec((1, BB, D), lambda p, t: (p, 0, 0)),
        ],
        out_specs=(blk_btd, blk_btd),
        scratch_shapes=[pltpu.VMEM((BB, D), jnp.float32)],
        compiler_params=pltpu.CompilerParams(
            dimension_semantics=("parallel", "arbitrary"),
        ),
        name="lif_scan",
    )(a, b, thr2d, h0_blocked)
    return h_post, s


# speedup vs baseline: 1.0059x; 1.0059x over previous
import jax
import jax.numpy as jnp
from jax.experimental import pallas as pl
from jax.experimental.pallas import tpu as pltpu

BT = 128
BB = 8


def _scan_body(a_ref, b_ref, thr_ref, h0_ref, h_out_ref, s_out_ref, h_scr):
    t_blk = pl.program_id(1)

    @pl.when(t_blk == 0)
    def _():
        h_scr[...] = h0_ref[0]

    thr = thr_ref[...]
    h = h_scr[...]
    for i in range(BT):
        h_pre = a_ref[:, i, :] * h + b_ref[:, i, :]
        spike = h_pre > thr
        h = jnp.where(spike, 0.0, h_pre)
        h_out_ref[:, i, :] = h
        s_out_ref[:, i, :] = jnp.where(spike, 1.0, 0.0)
    h_scr[...] = h


def kernel(a, b, threshold, h0):
    B, T, D = a.shape
    thr2d = threshold.reshape(1, D)
    h0_blocked = h0.reshape(B // BB, BB, D)

    grid = (B // BB, T // BT)
    blk_btd = pl.BlockSpec((BB, BT, D), lambda p, t: (p, t, 0))

    h_post, s = pl.pallas_call(
        _scan_body,
        out_shape=(
            jax.ShapeDtypeStruct((B, T, D), jnp.float32),
            jax.ShapeDtypeStruct((B, T, D), jnp.float32),
        ),
        grid=grid,
        in_specs=[
            blk_btd,
            blk_btd,
            pl.BlockSpec((1, D), lambda p, t: (0, 0)),
            pl.BlockSpec((1, BB, D), lambda p, t: (p, 0, 0)),
        ],
        out_specs=(blk_btd, blk_btd),
        scratch_shapes=[pltpu.VMEM((BB, D), jnp.float32)],
        compiler_params=pltpu.CompilerParams(
            dimension_semantics=("parallel", "arbitrary"),
        ),
        name="lif_scan",
    )(a, b, thr2d, h0_blocked)
    return h_post, s
